# baseline (speedup 1.0000x reference)
import functools

import jax
import jax.numpy as jnp
from jax import lax
from jax.experimental import pallas as pl
from jax.experimental.pallas import tpu as pltpu
from jax.experimental.pallas import tpu_sc as plsc

N = 10000
E = 320000
DH = 128
DC = 64
NC = 2
NS = 16
CH = 128
NCHUNK = 160
EP = NS * NCHUNK * CH
NP = 10112
RPT = NP // NS
DUMMY = N
ZR = 128
HCH = NCHUNK // 2

_MESH = dict(core_axis_name="c", subcore_axis_name="s")


def _sc_degree(dst_w):
    @functools.partial(
        pl.kernel,
        mesh=plsc.VectorSubcoreMesh(**_MESH),
        compiler_params=pltpu.CompilerParams(use_tc_tiling_on_sc=False),
        out_type=jax.ShapeDtypeStruct((NC, NP, 16), jnp.float32),
        scratch_types=[
            pltpu.VMEM((NCHUNK, CH), jnp.int32),
            pltpu.VMEM((CH, 16), jnp.float32),
            pltpu.VMEM((ZR, 16), jnp.float32),
            pltpu.VMEM_SHARED((NP, 16), jnp.float32),
            pltpu.SemaphoreType.DMA,
        ],
    )
    def k(dst_h, out_h, dst_v, ones_v, zero_v, acc_s, sem_d):
        c = lax.axis_index("c")
        s = lax.axis_index("s")

        def fill(i, _):
            ones_v[i, pl.ds(0, 16)] = jnp.ones((16,), jnp.float32)
            zero_v[i, pl.ds(0, 16)] = jnp.zeros((16,), jnp.float32)
            return 0

        lax.fori_loop(0, ZR, fill, 0)

        base = s * RPT
        for rep in range(4):
            pltpu.sync_copy(zero_v.at[pl.ds(0, ZR)],
                            acc_s.at[pl.ds(base + rep * ZR, ZR)])
        pltpu.sync_copy(zero_v.at[pl.ds(0, RPT - 4 * ZR)],
                        acc_s.at[pl.ds(base + 4 * ZR, RPT - 4 * ZR)])
        pltpu.sync_copy(dst_h.at[s], dst_v)
        plsc.subcore_barrier()

        def body(j, _):
            pltpu.async_copy(ones_v, acc_s.at[dst_v.at[j]], sem_d, add=True)
            return 0

        lax.fori_loop(c * HCH, c * HCH + HCH, body, 0)

        def drain(j, _):
            pltpu.make_async_copy(ones_v, acc_s.at[dst_v.at[j]], sem_d).wait()
            return 0

        lax.fori_loop(c * HCH, c * HCH + HCH, drain, 0)
        plsc.subcore_barrier()
        pltpu.sync_copy(acc_s.at[pl.ds(base, RPT)],
                        out_h.at[c, pl.ds(base, RPT)])

    return k(dst_w)


def _sc_prop(table, src_w, dst_w):
    @functools.partial(
        pl.kernel,
        mesh=plsc.VectorSubcoreMesh(**_MESH),
        compiler_params=pltpu.CompilerParams(use_tc_tiling_on_sc=False),
        out_type=jax.ShapeDtypeStruct((NC, NP, DC), jnp.float32),
        scratch_types=[
            pltpu.VMEM((NCHUNK, CH), jnp.int32),
            pltpu.VMEM((NCHUNK, CH), jnp.int32),
            pltpu.VMEM((2, CH, DC), jnp.float32),
            pltpu.VMEM((ZR, DC), jnp.float32),
            pltpu.VMEM_SHARED((NP, DC), jnp.float32),
            pltpu.SemaphoreType.DMA,
            pltpu.SemaphoreType.DMA,
        ],
    )
    def k(tab_h, src_h, dst_h, out_h, src_v, dst_v, rows_v, zero_v, acc_s,
          sem0, sem1):
        c = lax.axis_index("c")
        s = lax.axis_index("s")
        tab_c = tab_h.at[c]

        def fill(i, _):
            for jj in range(DC // 16):
                zero_v[i, pl.ds(jj * 16, 16)] = jnp.zeros((16,), jnp.float32)
            return 0

        lax.fori_loop(0, ZR, fill, 0)

        base = s * RPT
        for rep in range(4):
            pltpu.sync_copy(zero_v.at[pl.ds(0, ZR)],
                            acc_s.at[pl.ds(base + rep * ZR, ZR)])
        pltpu.sync_copy(zero_v.at[pl.ds(0, RPT - 4 * ZR)],
                        acc_s.at[pl.ds(base + 4 * ZR, RPT - 4 * ZR)])

        pltpu.sync_copy(src_h.at[s], src_v)
        pltpu.sync_copy(dst_h.at[s], dst_v)
        plsc.subcore_barrier()

        sems = (sem0, sem1)

        def start_gather(j, t):
            pltpu.make_async_copy(tab_c.at[src_v.at[j]], rows_v.at[t],
                                  sems[t]).start()

        def wait_gather(j, t):
            pltpu.make_async_copy(tab_c.at[src_v.at[j]], rows_v.at[t],
                                  sems[t]).wait()

        start_gather(0, 0)

        def body(jj, _):
            j0 = jj * 2
            wait_gather(j0, 0)
            start_gather(j0 + 1, 1)
            pltpu.sync_copy(rows_v.at[0], acc_s.at[dst_v.at[j0]], add=True)
            wait_gather(j0 + 1, 1)

            @pl.when(j0 + 2 < NCHUNK)
            def _():
                start_gather(j0 + 2, 0)

            pltpu.sync_copy(rows_v.at[1], acc_s.at[dst_v.at[j0 + 1]], add=True)
            return 0

        lax.fori_loop(0, NCHUNK // 2, body, 0)
        plsc.subcore_barrier()
        pltpu.sync_copy(acc_s.at[pl.ds(base, RPT)],
                        out_h.at[c, pl.ds(base, RPT)])

    return k(table, src_w, dst_w)


_GRID = 10
_BR = N // _GRID


def _row_spec(width):
    return pl.BlockSpec((_BR, width), lambda i: (i, 0))


def _half_spec(width):
    return pl.BlockSpec((NC, _BR, width), lambda i: (0, i, 0))


def _full_spec(a, b):
    return pl.BlockSpec((a, b), lambda i: (0, 0))


def _cat(r_r):
    return jnp.concatenate([r_r[0], r_r[1]], axis=1)


def _split_store(u_r, val):
    u_r[0] = val[:, :DC]
    u_r[1] = val[:, DC:]


_U_SHAPE = jax.ShapeDtypeStruct((NC, N, DC), jnp.float32)


def _tc_pre(x, w0, w1, b, rd):
    def body(x_r, w0_r, w1_r, b_r, rd_r, t0_r, u_r, dis_r):
        deg = rd_r[0, :, 0:1] + rd_r[1, :, 0:1]
        dis = jnp.where(deg > 0,
                        lax.rsqrt(jnp.maximum(deg, 1e-12)),
                        jnp.zeros_like(deg))
        xb = x_r[...]
        t0_r[...] = jnp.dot(xb, w0_r[...],
                            preferred_element_type=jnp.float32) + b_r[...]
        _split_store(u_r, dis * jnp.dot(xb, w1_r[...],
                                        preferred_element_type=jnp.float32))
        dis_r[...] = dis

    din = x.shape[1]
    return pl.pallas_call(
        body,
        grid=(_GRID,),
        in_specs=[_row_spec(din), _full_spec(din, DH), _full_spec(din, DH),
                  pl.BlockSpec((DH,), lambda i: (0,)), _half_spec(16)],
        out_specs=[_row_spec(DH), _half_spec(DC), _row_spec(1)],
        out_shape=[jax.ShapeDtypeStruct((N, DH), jnp.float32),
                   _U_SHAPE,
                   jax.ShapeDtypeStruct((N, 1), jnp.float32)],
    )(x, w0, w1, b, rd)


def _tc_fe_finish(r, t0, dis, w0, b):
    def body(r_r, t0_r, dis_r, w0_r, b_r, acc_r, u_r):
        dis = dis_r[...]
        h = t0_r[...] + dis * _cat(r_r)
        acc_r[...] = jnp.dot(h, w0_r[...],
                             preferred_element_type=jnp.float32) + b_r[...]
        _split_store(u_r, dis * h)

    return pl.pallas_call(
        body,
        grid=(_GRID,),
        in_specs=[_half_spec(DC), _row_spec(DH), _row_spec(1),
                  _full_spec(DH, DH), pl.BlockSpec((DH,), lambda i: (0,))],
        out_specs=[_row_spec(DH), _half_spec(DC)],
        out_shape=[jax.ShapeDtypeStruct((N, DH), jnp.float32), _U_SHAPE],
    )(r, t0, dis, w0, b)


def _tc_mid(r, acc, dis, wk):
    def body(r_r, acc_r, dis_r, wk_r, out_r, u_r):
        dis = dis_r[...]
        hk = dis * _cat(r_r)
        out_r[...] = acc_r[...] + jnp.dot(hk, wk_r[...],
                                          preferred_element_type=jnp.float32)
        _split_store(u_r, dis * hk)

    return pl.pallas_call(
        body,
        grid=(_GRID,),
        in_specs=[_half_spec(DC), _row_spec(DH), _row_spec(1),
                  _full_spec(DH, DH)],
        out_specs=[_row_spec(DH), _half_spec(DC)],
        out_shape=[jax.ShapeDtypeStruct((N, DH), jnp.float32), _U_SHAPE],
    )(r, acc, dis, wk)


def _tc_conv_boundary(r, acc, dis, w3, w0n, bn):
    def body(r_r, acc_r, dis_r, w3_r, w0_r, b_r, acc2_r, u_r):
        dis = dis_r[...]
        hk = dis * _cat(r_r)
        h = jax.nn.relu(acc_r[...] + jnp.dot(
            hk, w3_r[...], preferred_element_type=jnp.float32))
        acc2_r[...] = jnp.dot(h, w0_r[...],
                              preferred_element_type=jnp.float32) + b_r[...]
        _split_store(u_r, dis * h)

    return pl.pallas_call(
        body,
        grid=(_GRID,),
        in_specs=[_half_spec(DC), _row_spec(DH), _row_spec(1),
                  _full_spec(DH, DH), _full_spec(DH, DH),
                  pl.BlockSpec((DH,), lambda i: (0,))],
        out_specs=[_row_spec(DH), _half_spec(DC)],
        out_shape=[jax.ShapeDtypeStruct((N, DH), jnp.float32), _U_SHAPE],
    )(r, acc, dis, w3, w0n, bn)


def _tc_final(r, acc, dis, w3, fcw, fcb):
    def body(r_r, acc_r, dis_r, w3_r, fcw_r, fcb_r, out_r):
        dis = dis_r[...]
        hk = dis * _cat(r_r)
        h = acc_r[...] + jnp.dot(hk, w3_r[...],
                                 preferred_element_type=jnp.float32)
        out_r[...] = jnp.dot(h, fcw_r[...],
                             preferred_element_type=jnp.float32) + fcb_r[...]

    return pl.pallas_call(
        body,
        grid=(_GRID,),
        in_specs=[_half_spec(DC), _row_spec(DH), _row_spec(1),
                  _full_spec(DH, DH), _full_spec(DH, 1),
                  pl.BlockSpec((1,), lambda i: (0,))],
        out_specs=_row_spec(1),
        out_shape=jax.ShapeDtypeStruct((N, 1), jnp.float32),
    )(r, acc, dis, w3, fcw, fcb)


def kernel(x, edge_index, fe_W, fe_b, c1_W, c1_b, c2_W, c2_b, fc_W, fc_b):
    src = edge_index[0].astype(jnp.int32)
    dst = edge_index[1].astype(jnp.int32)
    pad = EP - E
    src_w = jnp.concatenate(
        [src, jnp.zeros((pad,), jnp.int32)]).reshape(NS, NCHUNK, CH)
    dst_w = jnp.concatenate(
        [dst, jnp.full((pad,), DUMMY, jnp.int32)]).reshape(NS, NCHUNK, CH)

    rd = _sc_degree(dst_w)
    t0, u, dis = _tc_pre(x, fe_W[0], fe_W[1], fe_b, rd)

    r = _sc_prop(u, src_w, dst_w)
    acc, u = _tc_fe_finish(r, t0, dis, c1_W[0], c1_b)
    for k in (1, 2):
        r = _sc_prop(u, src_w, dst_w)
        acc, u = _tc_mid(r, acc, dis, c1_W[k])
    r = _sc_prop(u, src_w, dst_w)
    acc, u = _tc_conv_boundary(r, acc, dis, c1_W[3], c2_W[0], c2_b)
    for k in (1, 2):
        r = _sc_prop(u, src_w, dst_w)
        acc, u = _tc_mid(r, acc, dis, c2_W[k])
    r = _sc_prop(u, src_w, dst_w)
    return _tc_final(r, acc, dis, c2_W[3], fc_W, fc_b)


# speedup vs baseline: 1.3586x; 1.3586x over previous
import functools

import jax
import jax.numpy as jnp
from jax import lax
from jax.experimental import pallas as pl
from jax.experimental.pallas import tpu as pltpu
from jax.experimental.pallas import tpu_sc as plsc

N = 10000
E = 320000
DH = 128
DC = 64
NC = 2
NS = 16
CH = 128
NCHUNK = 158
EP = NS * NCHUNK * CH
NP = 10112
RPT = NP // NS
DUMMY = N
ZR = 128
HCH = NCHUNK // 2

_MESH = dict(core_axis_name="c", subcore_axis_name="s")


def _sc_degree(dst_w):
    @functools.partial(
        pl.kernel,
        mesh=plsc.VectorSubcoreMesh(**_MESH),
        compiler_params=pltpu.CompilerParams(use_tc_tiling_on_sc=False),
        out_type=jax.ShapeDtypeStruct((NC, NP, 16), jnp.float32),
        scratch_types=[
            pltpu.VMEM((NCHUNK, CH), jnp.int32),
            pltpu.VMEM((CH, 16), jnp.float32),
            pltpu.VMEM((ZR, 16), jnp.float32),
            pltpu.VMEM_SHARED((NP, 16), jnp.float32),
            pltpu.SemaphoreType.DMA,
        ],
    )
    def k(dst_h, out_h, dst_v, ones_v, zero_v, acc_s, sem_d):
        c = lax.axis_index("c")
        s = lax.axis_index("s")

        def fill(i, _):
            ones_v[i, pl.ds(0, 16)] = jnp.ones((16,), jnp.float32)
            zero_v[i, pl.ds(0, 16)] = jnp.zeros((16,), jnp.float32)
            return 0

        lax.fori_loop(0, ZR, fill, 0)

        base = s * RPT
        for rep in range(4):
            pltpu.sync_copy(zero_v.at[pl.ds(0, ZR)],
                            acc_s.at[pl.ds(base + rep * ZR, ZR)])
        pltpu.sync_copy(zero_v.at[pl.ds(0, RPT - 4 * ZR)],
                        acc_s.at[pl.ds(base + 4 * ZR, RPT - 4 * ZR)])
        pltpu.sync_copy(dst_h.at[s], dst_v)
        plsc.subcore_barrier()

        def body(j, _):
            pltpu.async_copy(ones_v, acc_s.at[dst_v.at[j]], sem_d, add=True)
            return 0

        lax.fori_loop(c * HCH, c * HCH + HCH, body, 0)

        def drain(j, _):
            pltpu.make_async_copy(ones_v, acc_s.at[dst_v.at[j]], sem_d).wait()
            return 0

        lax.fori_loop(c * HCH, c * HCH + HCH, drain, 0)
        plsc.subcore_barrier()
        pltpu.sync_copy(acc_s.at[pl.ds(base, RPT)],
                        out_h.at[c, pl.ds(base, RPT)])

    return k(dst_w)


def _sc_prop(table, src_w, dst_w):
    @functools.partial(
        pl.kernel,
        mesh=plsc.VectorSubcoreMesh(**_MESH),
        compiler_params=pltpu.CompilerParams(use_tc_tiling_on_sc=False),
        out_type=jax.ShapeDtypeStruct((NC, NP, DC), jnp.float32),
        scratch_types=[
            pltpu.VMEM((NCHUNK, CH), jnp.int32),
            pltpu.VMEM((NCHUNK, CH), jnp.int32),
            pltpu.VMEM((2, CH, DC), jnp.float32),
            pltpu.VMEM((ZR, DC), jnp.float32),
            pltpu.VMEM_SHARED((NP, DC), jnp.float32),
            pltpu.SemaphoreType.DMA,
            pltpu.SemaphoreType.DMA,
        ],
    )
    def k(tab_h, src_h, dst_h, out_h, src_v, dst_v, rows_v, zero_v, acc_s,
          sem0, sem1):
        c = lax.axis_index("c")
        s = lax.axis_index("s")
        tab_c = tab_h.at[c]

        def fill(i, _):
            for jj in range(DC // 16):
                zero_v[i, pl.ds(jj * 16, 16)] = jnp.zeros((16,), jnp.float32)
            return 0

        lax.fori_loop(0, ZR, fill, 0)

        base = s * RPT
        for rep in range(4):
            pltpu.sync_copy(zero_v.at[pl.ds(0, ZR)],
                            acc_s.at[pl.ds(base + rep * ZR, ZR)])
        pltpu.sync_copy(zero_v.at[pl.ds(0, RPT - 4 * ZR)],
                        acc_s.at[pl.ds(base + 4 * ZR, RPT - 4 * ZR)])

        pltpu.sync_copy(src_h.at[s], src_v)
        pltpu.sync_copy(dst_h.at[s], dst_v)
        plsc.subcore_barrier()

        sems = (sem0, sem1)

        def start_gather(j, t):
            pltpu.make_async_copy(tab_c.at[src_v.at[j]], rows_v.at[t],
                                  sems[t]).start()

        def wait_gather(j, t):
            pltpu.make_async_copy(tab_c.at[src_v.at[j]], rows_v.at[t],
                                  sems[t]).wait()

        start_gather(0, 0)

        def body(jj, _):
            j0 = jj * 2
            wait_gather(j0, 0)
            start_gather(j0 + 1, 1)
            pltpu.sync_copy(rows_v.at[0], acc_s.at[dst_v.at[j0]], add=True)
            wait_gather(j0 + 1, 1)

            @pl.when(j0 + 2 < NCHUNK)
            def _():
                start_gather(j0 + 2, 0)

            pltpu.sync_copy(rows_v.at[1], acc_s.at[dst_v.at[j0 + 1]], add=True)
            return 0

        lax.fori_loop(0, NCHUNK // 2, body, 0)
        plsc.subcore_barrier()
        pltpu.sync_copy(acc_s.at[pl.ds(base, RPT)],
                        out_h.at[c, pl.ds(base, RPT)])

    return k(table, src_w, dst_w)


_GRID = 10
_BR = N // _GRID


def _row_spec(width):
    return pl.BlockSpec((_BR, width), lambda i: (i, 0))


def _half_spec(width):
    return pl.BlockSpec((NC, _BR, width), lambda i: (0, i, 0))


def _full_spec(a, b):
    return pl.BlockSpec((a, b), lambda i: (0, 0))


def _cat(r_r):
    return jnp.concatenate([r_r[0], r_r[1]], axis=1)


def _split_store(u_r, val):
    u_r[0] = val[:, :DC]
    u_r[1] = val[:, DC:]


_U_SHAPE = jax.ShapeDtypeStruct((NC, N, DC), jnp.float32)


def _tc_pre(x, w0, w1, b, rd):
    def body(x_r, w0_r, w1_r, b_r, rd_r, t0_r, u_r, dis_r):
        deg = rd_r[0, :, 0:1] + rd_r[1, :, 0:1]
        dis = jnp.where(deg > 0,
                        lax.rsqrt(jnp.maximum(deg, 1e-12)),
                        jnp.zeros_like(deg))
        xb = x_r[...]
        t0_r[...] = jnp.dot(xb, w0_r[...],
                            preferred_element_type=jnp.float32) + b_r[...]
        _split_store(u_r, dis * jnp.dot(xb, w1_r[...],
                                        preferred_element_type=jnp.float32))
        dis_r[...] = dis

    din = x.shape[1]
    return pl.pallas_call(
        body,
        grid=(_GRID,),
        in_specs=[_row_spec(din), _full_spec(din, DH), _full_spec(din, DH),
                  pl.BlockSpec((DH,), lambda i: (0,)), _half_spec(16)],
        out_specs=[_row_spec(DH), _half_spec(DC), _row_spec(1)],
        out_shape=[jax.ShapeDtypeStruct((N, DH), jnp.float32),
                   _U_SHAPE,
                   jax.ShapeDtypeStruct((N, 1), jnp.float32)],
    )(x, w0, w1, b, rd)


def _tc_fe_finish(r, t0, dis, w0, b):
    def body(r_r, t0_r, dis_r, w0_r, b_r, acc_r, u_r):
        dis = dis_r[...]
        h = t0_r[...] + dis * _cat(r_r)
        acc_r[...] = jnp.dot(h, w0_r[...],
                             preferred_element_type=jnp.float32) + b_r[...]
        _split_store(u_r, dis * h)

    return pl.pallas_call(
        body,
        grid=(_GRID,),
        in_specs=[_half_spec(DC), _row_spec(DH), _row_spec(1),
                  _full_spec(DH, DH), pl.BlockSpec((DH,), lambda i: (0,))],
        out_specs=[_row_spec(DH), _half_spec(DC)],
        out_shape=[jax.ShapeDtypeStruct((N, DH), jnp.float32), _U_SHAPE],
    )(r, t0, dis, w0, b)


def _tc_mid(r, acc, dis, wk):
    def body(r_r, acc_r, dis_r, wk_r, out_r, u_r):
        dis = dis_r[...]
        hk = dis * _cat(r_r)
        out_r[...] = acc_r[...] + jnp.dot(hk, wk_r[...],
                                          preferred_element_type=jnp.float32)
        _split_store(u_r, dis * hk)

    return pl.pallas_call(
        body,
        grid=(_GRID,),
        in_specs=[_half_spec(DC), _row_spec(DH), _row_spec(1),
                  _full_spec(DH, DH)],
        out_specs=[_row_spec(DH), _half_spec(DC)],
        out_shape=[jax.ShapeDtypeStruct((N, DH), jnp.float32), _U_SHAPE],
    )(r, acc, dis, wk)


def _tc_conv_boundary(r, acc, dis, w3, w0n, bn):
    def body(r_r, acc_r, dis_r, w3_r, w0_r, b_r, acc2_r, u_r):
        dis = dis_r[...]
        hk = dis * _cat(r_r)
        h = jax.nn.relu(acc_r[...] + jnp.dot(
            hk, w3_r[...], preferred_element_type=jnp.float32))
        acc2_r[...] = jnp.dot(h, w0_r[...],
                              preferred_element_type=jnp.float32) + b_r[...]
        _split_store(u_r, dis * h)

    return pl.pallas_call(
        body,
        grid=(_GRID,),
        in_specs=[_half_spec(DC), _row_spec(DH), _row_spec(1),
                  _full_spec(DH, DH), _full_spec(DH, DH),
                  pl.BlockSpec((DH,), lambda i: (0,))],
        out_specs=[_row_spec(DH), _half_spec(DC)],
        out_shape=[jax.ShapeDtypeStruct((N, DH), jnp.float32), _U_SHAPE],
    )(r, acc, dis, w3, w0n, bn)


def _tc_final(r, acc, dis, w3, fcw, fcb):
    def body(r_r, acc_r, dis_r, w3_r, fcw_r, fcb_r, out_r):
        dis = dis_r[...]
        hk = dis * _cat(r_r)
        h = acc_r[...] + jnp.dot(hk, w3_r[...],
                                 preferred_element_type=jnp.float32)
        out_r[...] = jnp.dot(h, fcw_r[...],
                             preferred_element_type=jnp.float32) + fcb_r[...]

    return pl.pallas_call(
        body,
        grid=(_GRID,),
        in_specs=[_half_spec(DC), _row_spec(DH), _row_spec(1),
                  _full_spec(DH, DH), _full_spec(DH, 1),
                  pl.BlockSpec((1,), lambda i: (0,))],
        out_specs=_row_spec(1),
        out_shape=jax.ShapeDtypeStruct((N, 1), jnp.float32),
    )(r, acc, dis, w3, fcw, fcb)


def kernel(x, edge_index, fe_W, fe_b, c1_W, c1_b, c2_W, c2_b, fc_W, fc_b):
    src = edge_index[0].astype(jnp.int32)
    dst = edge_index[1].astype(jnp.int32)
    pad = EP - E
    src_w = jnp.concatenate(
        [src, jnp.zeros((pad,), jnp.int32)]).reshape(NS, NCHUNK, CH)
    pad_dst = DUMMY + jnp.arange(pad, dtype=jnp.int32) % (NP - N)
    dst_w = jnp.concatenate([dst, pad_dst]).reshape(NS, NCHUNK, CH)

    rd = _sc_degree(dst_w)
    t0, u, dis = _tc_pre(x, fe_W[0], fe_W[1], fe_b, rd)

    r = _sc_prop(u, src_w, dst_w)
    acc, u = _tc_fe_finish(r, t0, dis, c1_W[0], c1_b)
    for k in (1, 2):
        r = _sc_prop(u, src_w, dst_w)
        acc, u = _tc_mid(r, acc, dis, c1_W[k])
    r = _sc_prop(u, src_w, dst_w)
    acc, u = _tc_conv_boundary(r, acc, dis, c1_W[3], c2_W[0], c2_b)
    for k in (1, 2):
        r = _sc_prop(u, src_w, dst_w)
        acc, u = _tc_mid(r, acc, dis, c2_W[k])
    r = _sc_prop(u, src_w, dst_w)
    return _tc_final(r, acc, dis, c2_W[3], fc_W, fc_b)
